# Initial kernel scaffold; baseline (speedup 1.0000x reference)
import jax
import jax.numpy as jnp
from jax.experimental import pallas as pl


def kernel(x, y, W):
    raise NotImplementedError("write your pallas kernel here")


# speedup vs baseline: 14.9963x; 14.9963x over previous
import jax
import jax.numpy as jnp
from jax.experimental import pallas as pl
from jax.experimental.pallas import tpu as pltpu

_N = 8192
_D = 64
_H = 16
_K = 20
_B = 128


def _body(x_ref, wmat_ref, out_ref, f_ref, sqr_ref):
    i = pl.program_id(0)

    @pl.when(i == 0)
    def _init():
        f = jnp.dot(x_ref[...], wmat_ref[...].T,
                    preferred_element_type=jnp.float32)
        f_ref[...] = f
        sqr_ref[...] = jnp.sum(f * f, axis=1)[None, :]

    fb = f_ref[pl.ds(i * _B, _B), :]
    sq_row = sqr_ref[...]
    sq_col = sqr_ref[0, pl.ds(i * _B, _B)][:, None]

    dot = jnp.dot(fb, f_ref[...].T, preferred_element_type=jnp.float32)
    d2 = sq_col + sq_row - 2.0 * dot

    cur = d2
    big = jnp.float32(3.0e38)
    t = None
    for _ in range(_K):
        t = jnp.min(cur, axis=1, keepdims=True)
        cur = jnp.where(cur <= t, big, cur)
    mask = d2 <= t
    w = jnp.where(mask, jnp.exp(-jnp.maximum(d2, 0.0)), 0.0)
    z = jnp.sum(w, axis=1, keepdims=True)
    out_ref[...] = jnp.dot(w / z, x_ref[...],
                           preferred_element_type=jnp.float32)


def kernel(x, y, W):
    del y
    out = pl.pallas_call(
        _body,
        grid=(_N // _B,),
        in_specs=[
            pl.BlockSpec((_N, _D), lambda i: (0, 0)),
            pl.BlockSpec((_H, _D), lambda i: (0, 0)),
        ],
        out_specs=pl.BlockSpec((_B, _D), lambda i: (i, 0)),
        out_shape=jax.ShapeDtypeStruct((_N, _D), jnp.float32),
        scratch_shapes=[
            pltpu.VMEM((_N, _H), jnp.float32),
            pltpu.VMEM((1, _N), jnp.float32),
        ],
        compiler_params=pltpu.CompilerParams(
            dimension_semantics=("arbitrary",),
        ),
    )(x, W)
    return out


# speedup vs baseline: 21.7293x; 1.4490x over previous
import jax
import jax.numpy as jnp
from jax import lax
from jax.experimental import pallas as pl
from jax.experimental.pallas import tpu as pltpu

_N = 8192
_D = 64
_H = 16
_K = 20
_B = 256
_C = 128
_NC = _N // _C
_P = 5

_BIG = 3.0e38


def _full_threshold(d2):
    cur = d2
    t = None
    for _ in range(_K):
        t = jnp.min(cur, axis=1, keepdims=True)
        cur = jnp.where(cur <= t, _BIG, cur)
    return t


def _body(x_ref, wmat_ref, out_ref, f_ref, sqr_ref):
    i = pl.program_id(0)

    @pl.when(i == 0)
    def _init():
        f = jnp.dot(x_ref[...], wmat_ref[...].T,
                    preferred_element_type=jnp.float32)
        f_ref[...] = f
        sqr_ref[...] = jnp.sum(f * f, axis=1)[None, :]

    fb = f_ref[pl.ds(i * _B, _B), :]
    sq_row = sqr_ref[...]
    sq_col = sqr_ref[0, pl.ds(i * _B, _B)][:, None]

    dot = jnp.dot(fb, f_ref[...].T, preferred_element_type=jnp.float32)
    d2 = sq_col + sq_row - 2.0 * dot

    rs = d2.reshape(_B, _NC, _C)
    pool = []
    for _ in range(_P):
        m = jnp.min(rs, axis=1)
        pool.append(m)
        rs = jnp.where(rs <= m[:, None, :], _BIG, rs)
    pv = jnp.concatenate(pool, axis=1)

    t = _full_threshold(pv)

    cnt = jnp.sum((d2 <= t).astype(jnp.float32), axis=1, keepdims=True)
    ok = jnp.all(cnt == float(_K))
    t = lax.cond(ok, lambda: t, lambda: _full_threshold(d2))

    w = jnp.where(d2 <= t, jnp.exp(-jnp.maximum(d2, 0.0)), 0.0)
    z = jnp.sum(w, axis=1, keepdims=True)
    out_ref[...] = jnp.dot(w / z, x_ref[...],
                           preferred_element_type=jnp.float32)


def kernel(x, y, W):
    del y
    out = pl.pallas_call(
        _body,
        grid=(_N // _B,),
        in_specs=[
            pl.BlockSpec((_N, _D), lambda i: (0, 0)),
            pl.BlockSpec((_H, _D), lambda i: (0, 0)),
        ],
        out_specs=pl.BlockSpec((_B, _D), lambda i: (i, 0)),
        out_shape=jax.ShapeDtypeStruct((_N, _D), jnp.float32),
        scratch_shapes=[
            pltpu.VMEM((_N, _H), jnp.float32),
            pltpu.VMEM((1, _N), jnp.float32),
        ],
        compiler_params=pltpu.CompilerParams(
            dimension_semantics=("arbitrary",),
        ),
    )(x, W)
    return out


# speedup vs baseline: 26.0703x; 1.1998x over previous
import jax
import jax.numpy as jnp
from jax import lax
from jax.experimental import pallas as pl
from jax.experimental.pallas import tpu as pltpu

_N = 8192
_D = 64
_H = 16
_K = 20
_B = 256
_C = 128
_NC = _N // _C
_P = 6

_BIG = 3.0e38
_LOG2E = 1.4426950408889634


def _full_threshold(g):
    cur = g
    t = None
    for _ in range(_K):
        t = jnp.min(cur, axis=1, keepdims=True)
        cur = jnp.where(cur <= t, _BIG, cur)
    return t


def _body(x_ref, wmat_ref, out_ref, xa_ref, fa_ref, f_ref):
    i = pl.program_id(0)

    @pl.when(i == 0)
    def _init():
        f = jnp.dot(x_ref[...], wmat_ref[...].T,
                    preferred_element_type=jnp.float32)
        f_ref[...] = f
        ft = f.T
        sq = jnp.sum(ft * ft, axis=0, keepdims=True)
        fa_ref[...] = jnp.concatenate([-2.0 * ft, sq], axis=0)
        xa_ref[...] = jnp.concatenate(
            [x_ref[...], jnp.ones((_N, 1), jnp.float32)], axis=1)

    fb = f_ref[pl.ds(i * _B, _B), :]
    fb_aug = jnp.concatenate([fb, jnp.ones((_B, 1), jnp.float32)], axis=1)
    g = jnp.dot(fb_aug, fa_ref[...],
                preferred_element_type=jnp.float32)
    sq_col = fa_ref[_H, pl.ds(i * _B, _B)][:, None]

    rs = g.reshape(_B, _NC, _C)
    pool = []
    m = jnp.min(rs, axis=1)
    pool.append(m)
    for _ in range(_P - 1):
        m = jnp.min(jnp.where(rs > m[:, None, :], rs, _BIG), axis=1)
        pool.append(m)
    pv = jnp.concatenate(pool, axis=1)

    t = _full_threshold(pv)

    ok = jnp.all(pool[-1] > t)
    t = lax.cond(ok, lambda: t, lambda: _full_threshold(g))

    w = jnp.where(g <= t,
                  jnp.exp2(jnp.maximum(g + sq_col, 0.0) * (-_LOG2E)),
                  0.0)
    acc = jnp.dot(w, xa_ref[...], preferred_element_type=jnp.float32)
    out_ref[...] = acc[:, :_D] / acc[:, _D:_D + 1]


def kernel(x, y, W):
    del y
    out = pl.pallas_call(
        _body,
        grid=(_N // _B,),
        in_specs=[
            pl.BlockSpec((_N, _D), lambda i: (0, 0)),
            pl.BlockSpec((_H, _D), lambda i: (0, 0)),
        ],
        out_specs=pl.BlockSpec((_B, _D), lambda i: (i, 0)),
        out_shape=jax.ShapeDtypeStruct((_N, _D), jnp.float32),
        scratch_shapes=[
            pltpu.VMEM((_N, _D + 1), jnp.float32),
            pltpu.VMEM((_H + 1, _N), jnp.float32),
            pltpu.VMEM((_N, _H), jnp.float32),
        ],
        compiler_params=pltpu.CompilerParams(
            dimension_semantics=("arbitrary",),
        ),
    )(x, W)
    return out
